# Initial kernel scaffold; baseline (speedup 1.0000x reference)
import jax
import jax.numpy as jnp
from jax.experimental import pallas as pl


def kernel(x, edge_index, W1, b1, W2, b2):
    raise NotImplementedError("write your pallas kernel here")


# speedup vs baseline: 14.8512x; 14.8512x over previous
import functools

import jax
import jax.numpy as jnp
from jax import lax
from jax.experimental import pallas as pl
from jax.experimental.pallas import tpu as pltpu
from jax.experimental.pallas import tpu_sc as plsc

N = 10000
E = 320000
DIN = 128
DH = 128
DOUT = 64

NC = 2
NS = 16
NW = NC * NS
CHUNK = 128

NACC = 10240
ROWS_PER_TILE = NACC // NS
ZR = 128

EF = E + N
EP = ((EF + NW * CHUNK - 1) // (NW * CHUNK)) * (NW * CHUNK)
EPT = EP // NW
CPT = EPT // CHUNK

ROW_BLOCK = 1024
GRID = (N + ROW_BLOCK - 1) // ROW_BLOCK

_mesh = plsc.VectorSubcoreMesh(core_axis_name="c", subcore_axis_name="s")
_sc_params = pltpu.CompilerParams(use_tc_tiling_on_sc=False)


def _sc_degree(dst_full):
    @functools.partial(
        pl.kernel,
        out_type=jax.ShapeDtypeStruct((NC, NACC), jnp.float32),
        mesh=_mesh,
        scratch_types=[
            pltpu.VMEM_SHARED((NACC,), jnp.float32),
            pltpu.VMEM((CHUNK,), jnp.int32),
            pltpu.VMEM((CHUNK,), jnp.float32),
            pltpu.VMEM((ROWS_PER_TILE,), jnp.float32),
        ],
    )
    def deg_kernel(dst_hbm, out_hbm, acc_sh, idx_v, ones_v, zeros_v):
        c = lax.axis_index("c")
        s = lax.axis_index("s")
        wid = c * NS + s

        @pl.loop(0, ROWS_PER_TILE, step=16)
        def _(i):
            zeros_v[pl.ds(i, 16)] = jnp.zeros((16,), jnp.float32)

        @pl.loop(0, CHUNK, step=16)
        def _(i):
            ones_v[pl.ds(i, 16)] = jnp.ones((16,), jnp.float32)

        row0 = s * ROWS_PER_TILE
        pltpu.sync_copy(zeros_v, acc_sh.at[pl.ds(row0, ROWS_PER_TILE)])
        plsc.subcore_barrier()

        base = wid * EPT

        @pl.loop(0, CPT)
        def _(ch):
            pltpu.sync_copy(dst_hbm.at[pl.ds(base + ch * CHUNK, CHUNK)], idx_v)
            pltpu.sync_copy(ones_v, acc_sh.at[idx_v], add=True)

        plsc.subcore_barrier()
        pltpu.sync_copy(acc_sh.at[pl.ds(row0, ROWS_PER_TILE)],
                        out_hbm.at[c, pl.ds(row0, ROWS_PER_TILE)])

    return deg_kernel(dst_full)


def _sc_aggregate(src_full, dst_full, y, d):
    @functools.partial(
        pl.kernel,
        out_type=jax.ShapeDtypeStruct((NC, NACC, d), jnp.float32),
        mesh=_mesh,
        compiler_params=_sc_params,
        scratch_types=[
            pltpu.VMEM_SHARED((NACC, d), jnp.float32),
            pltpu.VMEM((CHUNK,), jnp.int32),
            pltpu.VMEM((CHUNK,), jnp.int32),
            pltpu.VMEM((CHUNK, d), jnp.float32),
            pltpu.VMEM((ZR, d), jnp.float32),
        ],
    )
    def agg_kernel(src_hbm, dst_hbm, y_hbm, out_hbm,
                   acc_sh, src_v, dst_v, rows_v, zeros_v):
        c = lax.axis_index("c")
        s = lax.axis_index("s")
        wid = c * NS + s

        @pl.loop(0, ZR)
        def _(r):
            for j in range(d // 16):
                zeros_v[r, pl.ds(j * 16, 16)] = jnp.zeros((16,), jnp.float32)

        row0 = s * ROWS_PER_TILE

        @pl.loop(0, ROWS_PER_TILE, step=ZR)
        def _(r):
            pltpu.sync_copy(zeros_v, acc_sh.at[pl.ds(row0 + r, ZR)])

        plsc.subcore_barrier()

        base = wid * EPT

        @pl.loop(0, CPT)
        def _(ch):
            e0 = base + ch * CHUNK
            pltpu.sync_copy(src_hbm.at[pl.ds(e0, CHUNK)], src_v)
            pltpu.sync_copy(dst_hbm.at[pl.ds(e0, CHUNK)], dst_v)
            pltpu.sync_copy(y_hbm.at[src_v], rows_v)
            pltpu.sync_copy(rows_v, acc_sh.at[dst_v], add=True)

        plsc.subcore_barrier()
        pltpu.sync_copy(acc_sh.at[pl.ds(row0, ROWS_PER_TILE)],
                        out_hbm.at[c, pl.ds(row0, ROWS_PER_TILE)])

    return agg_kernel(src_full, dst_full, y)


def _tc_matmul(x, w):
    dout = w.shape[1]

    def body(x_ref, w_ref, o_ref):
        o_ref[...] = jnp.dot(x_ref[...], w_ref[...],
                             preferred_element_type=jnp.float32)

    return pl.pallas_call(
        body,
        grid=(GRID,),
        in_specs=[
            pl.BlockSpec((ROW_BLOCK, x.shape[1]), lambda i: (i, 0)),
            pl.BlockSpec((w.shape[0], dout), lambda i: (0, 0)),
        ],
        out_specs=pl.BlockSpec((ROW_BLOCK, dout), lambda i: (i, 0)),
        out_shape=jax.ShapeDtypeStruct((N, dout), jnp.float32),
    )(x, w)


def _tc_scale(xw, deg_a, deg_b):
    def body(xw_ref, da_ref, db_ref, y_ref, dinv_ref):
        dinv = lax.rsqrt(jnp.maximum(da_ref[...] + db_ref[...], 1e-12))
        dinv_ref[...] = dinv
        y_ref[...] = xw_ref[...] * dinv

    return pl.pallas_call(
        body,
        grid=(GRID,),
        in_specs=[
            pl.BlockSpec((ROW_BLOCK, DH), lambda i: (i, 0)),
            pl.BlockSpec((ROW_BLOCK, 1), lambda i: (i, 0)),
            pl.BlockSpec((ROW_BLOCK, 1), lambda i: (i, 0)),
        ],
        out_specs=[
            pl.BlockSpec((ROW_BLOCK, DH), lambda i: (i, 0)),
            pl.BlockSpec((ROW_BLOCK, 1), lambda i: (i, 0)),
        ],
        out_shape=[
            jax.ShapeDtypeStruct((N, DH), jnp.float32),
            jax.ShapeDtypeStruct((N, 1), jnp.float32),
        ],
    )(xw, deg_a, deg_b)


def _tc_layer2_in(s_a, s_b, dinv, b1, w2):
    def body(sa_ref, sb_ref, dinv_ref, b1_ref, w2_ref, y2_ref):
        dinv = dinv_ref[...]
        h = jnp.maximum(dinv * (sa_ref[...] + sb_ref[...]) + b1_ref[...], 0.0)
        y2_ref[...] = dinv * jnp.dot(h, w2_ref[...],
                                     preferred_element_type=jnp.float32)

    return pl.pallas_call(
        body,
        grid=(GRID,),
        in_specs=[
            pl.BlockSpec((ROW_BLOCK, DH), lambda i: (i, 0)),
            pl.BlockSpec((ROW_BLOCK, DH), lambda i: (i, 0)),
            pl.BlockSpec((ROW_BLOCK, 1), lambda i: (i, 0)),
            pl.BlockSpec((1, DH), lambda i: (0, 0)),
            pl.BlockSpec((DH, DOUT), lambda i: (0, 0)),
        ],
        out_specs=pl.BlockSpec((ROW_BLOCK, DOUT), lambda i: (i, 0)),
        out_shape=jax.ShapeDtypeStruct((N, DOUT), jnp.float32),
    )(s_a, s_b, dinv, b1, w2)


def _tc_final(s_a, s_b, dinv, b2):
    def body(sa_ref, sb_ref, dinv_ref, b2_ref, o_ref):
        o_ref[...] = (dinv_ref[...] * (sa_ref[...] + sb_ref[...])
                      + b2_ref[...])

    return pl.pallas_call(
        body,
        grid=(GRID,),
        in_specs=[
            pl.BlockSpec((ROW_BLOCK, DOUT), lambda i: (i, 0)),
            pl.BlockSpec((ROW_BLOCK, DOUT), lambda i: (i, 0)),
            pl.BlockSpec((ROW_BLOCK, 1), lambda i: (i, 0)),
            pl.BlockSpec((1, DOUT), lambda i: (0, 0)),
        ],
        out_specs=pl.BlockSpec((ROW_BLOCK, DOUT), lambda i: (i, 0)),
        out_shape=jax.ShapeDtypeStruct((N, DOUT), jnp.float32),
    )(s_a, s_b, dinv, b2)


def kernel(x, edge_index, W1, b1, W2, b2):
    src = edge_index[0]
    dst = edge_index[1]
    loop = jnp.arange(N, dtype=jnp.int32)
    pad = EP - EF
    src_full = jnp.concatenate([src, loop, jnp.zeros((pad,), jnp.int32)])
    dst_full = jnp.concatenate([dst, loop, jnp.full((pad,), N, jnp.int32)])

    deg_p = _sc_degree(dst_full)
    deg_a = deg_p[0].reshape(NACC, 1)
    deg_b = deg_p[1].reshape(NACC, 1)

    xw1 = _tc_matmul(x, W1)
    y1, dinv = _tc_scale(xw1, deg_a, deg_b)

    s1 = _sc_aggregate(src_full, dst_full, y1, DH)
    y2 = _tc_layer2_in(s1[0], s1[1], dinv,
                       b1.reshape(1, DH), W2)

    s2 = _sc_aggregate(src_full, dst_full, y2, DOUT)
    return _tc_final(s2[0], s2[1], dinv, b2.reshape(1, DOUT))
